# Initial kernel scaffold; baseline (speedup 1.0000x reference)
import jax
import jax.numpy as jnp
from jax.experimental import pallas as pl


def kernel(h, edge_index, params):
    raise NotImplementedError("write your pallas kernel here")


# speedup vs baseline: 7.8016x; 7.8016x over previous
import functools

import jax
import jax.numpy as jnp
from jax import lax
from jax.experimental import pallas as pl
from jax.experimental.pallas import tpu as pltpu
from jax.experimental.pallas import tpu_sc as plsc

N = 10000
E = 320000
D_IN = 128
HID = 64

NC = 2
NS = 16
TILES = NC * NS
EPT = E // TILES
C = 100
NCH = EPT // C
ROWS = 624
TAIL = N - NS * ROWS

_mesh = plsc.VectorSubcoreMesh(core_axis_name="c", subcore_axis_name="s")


def _make_sc_segsum(D):
    @functools.partial(
        pl.kernel,
        out_type=jax.ShapeDtypeStruct((NC, N, D), jnp.float32),
        mesh=_mesh,
        compiler_params=pltpu.CompilerParams(use_tc_tiling_on_sc=False),
        scratch_types=[
            pltpu.VMEM((NCH, C), jnp.int32),
            pltpu.VMEM((NCH, C), jnp.int32),
            pltpu.VMEM((C, D), jnp.float32),
            pltpu.VMEM_SHARED((N, D), jnp.float32),
            pltpu.SemaphoreType.DMA,
        ],
    )
    def _sc_segsum(x_hbm, src_hbm, dst_hbm, zero_hbm, out_hbm,
                   src_v, dst_v, rows_v, acc_sh, sem):
        c = lax.axis_index("c")
        s = lax.axis_index("s")
        tid = c * NS + s
        pltpu.sync_copy(zero_hbm.at[pl.ds(s * ROWS, ROWS)],
                        acc_sh.at[pl.ds(s * ROWS, ROWS)])

        @pl.when(s == 0)
        def _():
            pltpu.sync_copy(zero_hbm.at[pl.ds(NS * ROWS, TAIL)],
                            acc_sh.at[pl.ds(NS * ROWS, TAIL)])

        pltpu.sync_copy(src_hbm.at[tid], src_v)
        pltpu.sync_copy(dst_hbm.at[tid], dst_v)
        plsc.subcore_barrier()

        def body(j, carry):
            pltpu.async_copy(x_hbm.at[src_v.at[j]], rows_v, sem).wait()
            pltpu.sync_copy(rows_v, acc_sh.at[dst_v.at[j]], add=True)
            return carry

        lax.fori_loop(0, NCH, body, 0)
        plsc.subcore_barrier()
        pltpu.sync_copy(acc_sh.at[pl.ds(s * ROWS, ROWS)],
                        out_hbm.at[c, pl.ds(s * ROWS, ROWS)])

        @pl.when(s == 0)
        def _():
            pltpu.sync_copy(acc_sh.at[pl.ds(NS * ROWS, TAIL)],
                            out_hbm.at[c, pl.ds(NS * ROWS, TAIL)])

    return _sc_segsum


_sc_segsum_in = _make_sc_segsum(D_IN)
_sc_segsum_hid = _make_sc_segsum(HID)


def _tc_layer(x, parts, w1, b1, w2, b2, gamma, beta):
    def body(x_ref, p_ref, w1_ref, b1_ref, w2_ref, b2_ref, g_ref, be_ref,
             o_ref):
        r = x_ref[...] + p_ref[0] + p_ref[1]
        r1 = jnp.maximum(
            jnp.dot(r, w1_ref[...], preferred_element_type=jnp.float32)
            + b1_ref[...], 0.0)
        r2 = jnp.dot(r1, w2_ref[...],
                     preferred_element_type=jnp.float32) + b2_ref[...]
        mean = jnp.mean(r2, axis=0, keepdims=True)
        d = r2 - mean
        var = jnp.mean(d * d, axis=0, keepdims=True)
        xn = g_ref[...] * d * lax.rsqrt(var + 1e-5) + be_ref[...]
        o_ref[...] = jnp.maximum(xn, 0.0)

    return pl.pallas_call(
        body,
        out_shape=jax.ShapeDtypeStruct((N, HID), jnp.float32),
    )(x, parts, w1, b1, w2, b2, gamma, beta)


def kernel(h, edge_index, params):
    src = edge_index[0].reshape(TILES, NCH, C)
    dst = edge_index[1].reshape(TILES, NCH, C)
    zero_in = jnp.zeros((N, D_IN), jnp.float32)
    zero_hid = jnp.zeros((N, HID), jnp.float32)
    b1 = [b.reshape(1, HID) for b in params["b1"]]
    b2 = [b.reshape(1, HID) for b in params["b2"]]
    gamma = [g.reshape(1, HID) for g in params["gamma"]]
    beta = [b.reshape(1, HID) for b in params["beta"]]

    x = h
    for l in range(4):
        if l == 0:
            parts = _sc_segsum_in(x, src, dst, zero_in)
        else:
            parts = _sc_segsum_hid(x, src, dst, zero_hid)
        x = _tc_layer(x, parts, params["W1"][l], b1[l], params["W2"][l],
                      b2[l], gamma[l], beta[l])
    return x
